# baseline (speedup 1.0000x reference)
import jax
import jax.numpy as jnp
from jax.experimental import pallas as pl
from jax.experimental.pallas import tpu as pltpu

N = 4096
BM = 256


def _support_kernel(x_ref, w_ref, o_ref):
    o_ref[...] = jnp.dot(
        x_ref[...], w_ref[...], preferred_element_type=jnp.float32
    ).astype(jnp.bfloat16)


def _agg_next_kernel(adj_ref, s_ref, b_ref, w_ref, o_ref):
    acc = jnp.dot(adj_ref[...], s_ref[...], preferred_element_type=jnp.float32)
    h = jnp.maximum(acc + b_ref[...], 0.0).astype(jnp.bfloat16)
    o_ref[...] = jnp.dot(
        h, w_ref[...], preferred_element_type=jnp.float32
    ).astype(jnp.bfloat16)


def _agg_final_kernel(adj_ref, s_ref, b_ref, o_ref):
    acc = jnp.dot(adj_ref[...], s_ref[...], preferred_element_type=jnp.float32)
    h = jnp.maximum(acc + b_ref[...], 0.0)
    m = jnp.max(h, axis=1, keepdims=True)
    lse = jnp.log(jnp.sum(jnp.exp(h - m), axis=1, keepdims=True)) + m
    o_ref[...] = h - lse


_PARAMS = pltpu.CompilerParams(dimension_semantics=("parallel",))


def _support(x16, w16):
    m, k = x16.shape
    kout = w16.shape[1]
    return pl.pallas_call(
        _support_kernel,
        grid=(m // 512,),
        in_specs=[
            pl.BlockSpec((512, k), lambda i: (i, 0)),
            pl.BlockSpec((k, kout), lambda i: (0, 0)),
        ],
        out_specs=pl.BlockSpec((512, kout), lambda i: (i, 0)),
        out_shape=jax.ShapeDtypeStruct((m, kout), jnp.bfloat16),
        compiler_params=_PARAMS,
    )(x16, w16)


def _agg_next(adj16, s, b, w16):
    k = s.shape[1]
    kout = w16.shape[1]
    return pl.pallas_call(
        _agg_next_kernel,
        grid=(N // BM,),
        in_specs=[
            pl.BlockSpec((BM, N), lambda i: (i, 0)),
            pl.BlockSpec((N, k), lambda i: (0, 0)),
            pl.BlockSpec((1, k), lambda i: (0, 0)),
            pl.BlockSpec((k, kout), lambda i: (0, 0)),
        ],
        out_specs=pl.BlockSpec((BM, kout), lambda i: (i, 0)),
        out_shape=jax.ShapeDtypeStruct((N, kout), jnp.bfloat16),
        compiler_params=_PARAMS,
    )(adj16, s, b, w16)


def _agg_final(adj16, s, b):
    k = s.shape[1]
    return pl.pallas_call(
        _agg_final_kernel,
        grid=(N // BM,),
        in_specs=[
            pl.BlockSpec((BM, N), lambda i: (i, 0)),
            pl.BlockSpec((N, k), lambda i: (0, 0)),
            pl.BlockSpec((1, k), lambda i: (0, 0)),
        ],
        out_specs=pl.BlockSpec((BM, k), lambda i: (i, 0)),
        out_shape=jax.ShapeDtypeStruct((N, k), jnp.float32),
        compiler_params=_PARAMS,
    )(adj16, s, b)


def kernel(x, adj, W1, b1, W2, b2, W3, b3, W4, b4):
    bf = jnp.bfloat16
    adj16 = adj.astype(bf)
    s1 = _support(x.astype(bf), W1.astype(bf))
    s2 = _agg_next(adj16, s1, b1.reshape(1, -1), W2.astype(bf))
    s3 = _agg_next(adj16, s2, b2.reshape(1, -1), W3.astype(bf))
    s4 = _agg_next(adj16, s3, b3.reshape(1, -1), W4.astype(bf))
    return _agg_final(adj16, s4, b4.reshape(1, -1))
